# baseline (speedup 1.0000x reference)
import functools

import jax
import jax.numpy as jnp
from jax import lax
from jax.experimental import pallas as pl
from jax.experimental.pallas import tpu as pltpu
from jax.experimental.pallas import tpu_sc as plsc

F32 = jnp.float32

_LANE = 128
_KB = 128
_TILES = 16
_WIN = 16
_BN = 1000
_EPS = 1e-5


def _npad(n):
    rpt = -(-(-(-n // _TILES)) // 8) * 8
    while rpt * _TILES <= n:
        rpt += 8
    return rpt * _TILES


@functools.lru_cache(maxsize=None)
def _make_agg(C, npad, nb):
    rpt = npad // _TILES
    cpc = C // 2
    mesh = plsc.VectorSubcoreMesh(core_axis_name="c", subcore_axis_name="s")

    def body(hflat, gidx, sidx, out, gidx_v, sidx_v, rows_v, acc, sem):
        cid = lax.axis_index("c")
        sid = lax.axis_index("s")
        r0 = sid * rpt
        pltpu.sync_copy(sidx.at[sid], sidx_v)
        for cc in range(cpc):
            c = cid * cpc + cc
            pltpu.sync_copy(hflat.at[pl.ds(c * npad + r0, rpt)],
                            acc.at[pl.ds(r0, rpt)])
            pltpu.sync_copy(gidx.at[c, sid], gidx_v)
            plsc.subcore_barrier()

            @pl.loop(0, nb)
            def _edges(j):
                pltpu.async_copy(hflat.at[gidx_v.at[j]], rows_v, sem).wait()
                pltpu.sync_copy(rows_v, acc.at[sidx_v.at[j]], add=True)

            plsc.subcore_barrier()
            pltpu.sync_copy(acc.at[pl.ds(r0, rpt)], out.at[c, pl.ds(r0, rpt)])
            plsc.subcore_barrier()

    return pl.kernel(
        body,
        out_type=jax.ShapeDtypeStruct((C, npad, _LANE), F32),
        mesh=mesh,
        scratch_types=[
            pltpu.VMEM((nb, _KB), jnp.int32),
            pltpu.VMEM((nb, _KB), jnp.int32),
            pltpu.VMEM((_KB, _LANE), F32),
            pltpu.VMEM_SHARED((npad, _LANE), F32),
            pltpu.SemaphoreType.DMA,
        ],
    )


def _acc_stats(ref, st, i):
    @pl.when(i == 0)
    def _():
        ref[...] = st

    @pl.when(i > 0)
    def _():
        ref[...] = ref[...] + st


def _k0(x, npad):
    n, din = x.shape
    C = din // _LANE
    nblk = n // _BN

    def body(x_ref, xc_ref, gp_ref):
        i = pl.program_id(0)
        xb = x_ref[...]
        for c in range(C):
            xc_ref[c] = xb[:, c * _LANE:(c + 1) * _LANE]
        st = jnp.concatenate(
            [jnp.sum(xb, axis=0)[None], jnp.zeros((7, din), F32)], axis=0)
        _acc_stats(gp_ref, st, i)

    return pl.pallas_call(
        body,
        grid=(nblk,),
        in_specs=[pl.BlockSpec((_BN, din), lambda i: (i, 0))],
        out_specs=[pl.BlockSpec((C, _BN, _LANE), lambda i: (0, i, 0)),
                   pl.BlockSpec((8, din), lambda i: (0, 0))],
        out_shape=[jax.ShapeDtypeStruct((C, npad, _LANE), F32),
                   jax.ShapeDtypeStruct((8, din), F32)],
    )(x)


def _s1(pooled, w1r, b1, n):
    C, _, wc = pooled.shape
    H = w1r.shape[2]
    nblk = n // _BN

    def body(p_ref, w_ref, b_ref, m_ref, st_ref):
        i = pl.program_id(0)
        acc = jnp.zeros((_BN, H), F32)
        for c in range(C):
            acc = acc + jnp.dot(p_ref[c].astype(jnp.bfloat16), w_ref[c],
                                preferred_element_type=F32)
        m = acc + b_ref[...]
        m_ref[...] = m
        st = jnp.concatenate(
            [jnp.sum(m, axis=0)[None], jnp.sum(m * m, axis=0)[None],
             jnp.zeros((6, H), F32)], axis=0)
        _acc_stats(st_ref, st, i)

    return pl.pallas_call(
        body,
        grid=(nblk,),
        in_specs=[pl.BlockSpec((C, _BN, wc), lambda i: (0, i, 0)),
                  pl.BlockSpec((C, wc, H), lambda i: (0, 0, 0)),
                  pl.BlockSpec((1, H), lambda i: (0, 0))],
        out_specs=[pl.BlockSpec((_BN, H), lambda i: (i, 0)),
                   pl.BlockSpec((8, H), lambda i: (0, 0))],
        out_shape=[jax.ShapeDtypeStruct((n, H), F32),
                   jax.ShapeDtypeStruct((8, H), F32)],
    )(pooled, w1r, b1)


def _bn_affine(st_ref, g_ref, b_ref, inv_n):
    mean = st_ref[0:1, :] * inv_n
    var = st_ref[1:2, :] * inv_n - mean * mean
    scale = g_ref[...] * lax.rsqrt(var + _EPS)
    shift = b_ref[...] - mean * scale
    return scale, shift


def _s2(m, st, g, b, w2, b2):
    n, H = m.shape
    nblk = n // _BN
    inv_n = 1.0 / n

    def body(m_ref, st_ref, g_ref, b_ref, w_ref, b2_ref, m2_ref, st2_ref):
        i = pl.program_id(0)
        scale, shift = _bn_affine(st_ref, g_ref, b_ref, inv_n)
        t = jnp.maximum(m_ref[...] * scale + shift, 0.0)
        m2 = jnp.dot(t.astype(jnp.bfloat16), w_ref[...],
                     preferred_element_type=F32) + b2_ref[...]
        m2_ref[...] = m2
        st = jnp.concatenate(
            [jnp.sum(m2, axis=0)[None], jnp.sum(m2 * m2, axis=0)[None],
             jnp.zeros((6, H), F32)], axis=0)
        _acc_stats(st2_ref, st, i)

    return pl.pallas_call(
        body,
        grid=(nblk,),
        in_specs=[pl.BlockSpec((_BN, H), lambda i: (i, 0)),
                  pl.BlockSpec((8, H), lambda i: (0, 0)),
                  pl.BlockSpec((1, H), lambda i: (0, 0)),
                  pl.BlockSpec((1, H), lambda i: (0, 0)),
                  pl.BlockSpec((H, H), lambda i: (0, 0)),
                  pl.BlockSpec((1, H), lambda i: (0, 0))],
        out_specs=[pl.BlockSpec((_BN, H), lambda i: (i, 0)),
                   pl.BlockSpec((8, H), lambda i: (0, 0))],
        out_shape=[jax.ShapeDtypeStruct((n, H), F32),
                   jax.ShapeDtypeStruct((8, H), F32)],
    )(m, st, g, b, w2, b2)


def _s3(m2, st2, g, b, npad):
    n, H = m2.shape
    C = H // _LANE
    nblk = n // _BN
    inv_n = 1.0 / n

    def body(m2_ref, st_ref, g_ref, b_ref, hc_ref, gp_ref):
        i = pl.program_id(0)
        scale, shift = _bn_affine(st_ref, g_ref, b_ref, inv_n)
        h = jnp.maximum(m2_ref[...] * scale + shift, 0.0)
        for c in range(C):
            hc_ref[c] = h[:, c * _LANE:(c + 1) * _LANE]
        st = jnp.concatenate(
            [jnp.sum(h, axis=0)[None], jnp.zeros((7, H), F32)], axis=0)
        _acc_stats(gp_ref, st, i)

    return pl.pallas_call(
        body,
        grid=(nblk,),
        in_specs=[pl.BlockSpec((_BN, H), lambda i: (i, 0)),
                  pl.BlockSpec((8, H), lambda i: (0, 0)),
                  pl.BlockSpec((1, H), lambda i: (0, 0)),
                  pl.BlockSpec((1, H), lambda i: (0, 0))],
        out_specs=[pl.BlockSpec((C, _BN, _LANE), lambda i: (0, i, 0)),
                   pl.BlockSpec((8, H), lambda i: (0, 0))],
        out_shape=[jax.ShapeDtypeStruct((C, npad, _LANE), F32),
                   jax.ShapeDtypeStruct((8, H), F32)],
    )(m2, st2, g, b)


def _head(gps, ws, bs):
    od = ws[0].shape[1]
    nl = len(ws)

    def body(*refs):
        gp_refs = refs[:nl]
        w_refs = refs[nl:2 * nl]
        b_refs = refs[2 * nl:3 * nl]
        out_ref = refs[3 * nl]
        tot = jnp.zeros((8, od), F32)
        bsum = jnp.zeros((1, od), F32)
        for gp, w, bb in zip(gp_refs, w_refs, b_refs):
            tot = tot + jnp.dot(gp[...], w[...], preferred_element_type=F32)
            bsum = bsum + bb[...]
        rows = lax.broadcasted_iota(jnp.int32, (8, od), 0)
        out_ref[...] = tot + jnp.where(rows == 0, bsum, 0.0)

    specs = ([pl.BlockSpec(g.shape, lambda i: (0, 0)) for g in gps]
             + [pl.BlockSpec(w.shape, lambda i: (0, 0)) for w in ws]
             + [pl.BlockSpec(b.shape, lambda i: (0, 0)) for b in bs])
    return pl.pallas_call(
        body,
        grid=(1,),
        in_specs=specs,
        out_specs=pl.BlockSpec((8, od), lambda i: (0, 0)),
        out_shape=jax.ShapeDtypeStruct((8, od), F32),
    )(*gps, *ws, *bs)


def _agg(hflat, gidx, sidx, C, npad, nb):
    return _make_agg(C, npad, nb)(hflat, gidx, sidx)


def kernel(x, edge_index, batch, params):
    n, din = x.shape
    e = edge_index.shape[1]
    convs = params['convs']
    preds = params['preds']

    nb = -(-e // (_TILES * _KB))
    epad = _TILES * nb * _KB
    npad = _npad(n)
    src = edge_index[0]
    dst = edge_index[1]
    dsta = jnp.concatenate(
        [dst, jnp.zeros((epad - e,), jnp.int32)]).reshape(_TILES, nb, _KB)
    sidx = jnp.concatenate(
        [src, jnp.full((epad - e,), n, jnp.int32)]).reshape(_TILES, nb, _KB)

    def gidx_for(C):
        offs = (jnp.arange(C, dtype=jnp.int32) * npad)[:, None, None, None]
        return dsta[None] + offs

    xc, gp0 = _k0(x, npad)
    gps = [gp0]
    hflat = xc.reshape(-1, _LANE)
    C = din // _LANE
    for cv in convs:
        pooled = _agg(hflat, gidx_for(C), sidx, C, npad, nb)
        hid = cv['W1'].shape[1]
        m, st = _s1(pooled,
                    cv['W1'].reshape(C, _LANE, hid).astype(jnp.bfloat16),
                    cv['b1'].reshape(1, -1), n)
        m2, st2 = _s2(m, st, cv['bn1_g'].reshape(1, -1),
                      cv['bn1_b'].reshape(1, -1),
                      cv['W2'].astype(jnp.bfloat16),
                      cv['b2'].reshape(1, -1))
        hc, gp = _s3(m2, st2, cv['bn_g'].reshape(1, -1),
                     cv['bn_b'].reshape(1, -1), npad)
        gps.append(gp)
        hflat = hc.reshape(-1, _LANE)
        C = hc.shape[0]

    score8 = _head(gps, [p['W'] for p in preds],
                   [p['b'].reshape(1, -1) for p in preds])
    return score8[0:1]


# speedup vs baseline: 1.0172x; 1.0172x over previous
import functools

import jax
import jax.numpy as jnp
from jax import lax
from jax.experimental import pallas as pl
from jax.experimental.pallas import tpu as pltpu
from jax.experimental.pallas import tpu_sc as plsc

F32 = jnp.float32

_LANE = 128
_KB = 128
_TILES = 16
_WIN = 16
_BN = 1000
_EPS = 1e-5


def _npad(n):
    rpt = -(-(-(-n // _TILES)) // 8) * 8
    while rpt * _TILES <= n:
        rpt += 8
    return rpt * _TILES


@functools.lru_cache(maxsize=None)
def _make_agg(C, npad, nb):
    rpt = npad // _TILES
    cpc = C // 2
    mesh = plsc.VectorSubcoreMesh(core_axis_name="c", subcore_axis_name="s")

    def body(hflat, gidx, sidx, out, gidx_v, sidx_v, rows_v, acc, sem):
        cid = lax.axis_index("c")
        sid = lax.axis_index("s")
        r0 = sid * rpt
        pltpu.sync_copy(sidx.at[sid], sidx_v)
        for cc in range(cpc):
            c = cid * cpc + cc
            pltpu.sync_copy(hflat.at[pl.ds(c * npad + r0, rpt)],
                            acc.at[pl.ds(r0, rpt)])
            pltpu.sync_copy(gidx.at[c, sid], gidx_v)
            plsc.subcore_barrier()

            @pl.loop(0, nb)
            def _edges(j):
                pltpu.async_copy(hflat.at[gidx_v.at[j]], rows_v, sem).wait()
                pltpu.sync_copy(rows_v, acc.at[sidx_v.at[j]], add=True)

            plsc.subcore_barrier()
            pltpu.sync_copy(acc.at[pl.ds(r0, rpt)], out.at[c, pl.ds(r0, rpt)])
            plsc.subcore_barrier()

    return pl.kernel(
        body,
        out_type=jax.ShapeDtypeStruct((C, npad, _LANE), F32),
        mesh=mesh,
        scratch_types=[
            pltpu.VMEM((nb, _KB), jnp.int32),
            pltpu.VMEM((nb, _KB), jnp.int32),
            pltpu.VMEM((_KB, _LANE), F32),
            pltpu.VMEM_SHARED((npad, _LANE), F32),
            pltpu.SemaphoreType.DMA,
        ],
    )


def _acc_stats(ref, st, i):
    @pl.when(i == 0)
    def _():
        ref[...] = st

    @pl.when(i > 0)
    def _():
        ref[...] = ref[...] + st


def _k0(x, npad):
    n, din = x.shape
    C = din // _LANE
    nblk = n // _BN

    def body(x_ref, xc_ref, gp_ref):
        i = pl.program_id(0)
        xb = x_ref[...]
        for c in range(C):
            xc_ref[c] = xb[:, c * _LANE:(c + 1) * _LANE]
        st = jnp.concatenate(
            [jnp.sum(xb, axis=0)[None], jnp.zeros((7, din), F32)], axis=0)
        _acc_stats(gp_ref, st, i)

    return pl.pallas_call(
        body,
        grid=(nblk,),
        in_specs=[pl.BlockSpec((_BN, din), lambda i: (i, 0))],
        out_specs=[pl.BlockSpec((C, _BN, _LANE), lambda i: (0, i, 0)),
                   pl.BlockSpec((8, din), lambda i: (0, 0))],
        out_shape=[jax.ShapeDtypeStruct((C, npad, _LANE), F32),
                   jax.ShapeDtypeStruct((8, din), F32)],
    )(x)


def _s1(pooled, w1r, b1, n):
    C, _, wc = pooled.shape
    H = w1r.shape[2]
    nblk = n // _BN

    def body(p_ref, w_ref, b_ref, m_ref, st_ref):
        i = pl.program_id(0)
        acc = jnp.zeros((_BN, H), F32)
        for c in range(C):
            acc = acc + jnp.dot(p_ref[c], w_ref[c], preferred_element_type=F32)
        m = acc + b_ref[...]
        m_ref[...] = m
        st = jnp.concatenate(
            [jnp.sum(m, axis=0)[None], jnp.sum(m * m, axis=0)[None],
             jnp.zeros((6, H), F32)], axis=0)
        _acc_stats(st_ref, st, i)

    return pl.pallas_call(
        body,
        grid=(nblk,),
        in_specs=[pl.BlockSpec((C, _BN, wc), lambda i: (0, i, 0)),
                  pl.BlockSpec((C, wc, H), lambda i: (0, 0, 0)),
                  pl.BlockSpec((1, H), lambda i: (0, 0))],
        out_specs=[pl.BlockSpec((_BN, H), lambda i: (i, 0)),
                   pl.BlockSpec((8, H), lambda i: (0, 0))],
        out_shape=[jax.ShapeDtypeStruct((n, H), F32),
                   jax.ShapeDtypeStruct((8, H), F32)],
    )(pooled, w1r, b1)


def _bn_affine(st_ref, g_ref, b_ref, inv_n):
    mean = st_ref[0:1, :] * inv_n
    var = st_ref[1:2, :] * inv_n - mean * mean
    scale = g_ref[...] * lax.rsqrt(var + _EPS)
    shift = b_ref[...] - mean * scale
    return scale, shift


def _s2(m, st, g, b, w2, b2):
    n, H = m.shape
    nblk = n // _BN
    inv_n = 1.0 / n

    def body(m_ref, st_ref, g_ref, b_ref, w_ref, b2_ref, m2_ref, st2_ref):
        i = pl.program_id(0)
        scale, shift = _bn_affine(st_ref, g_ref, b_ref, inv_n)
        t = jnp.maximum(m_ref[...] * scale + shift, 0.0)
        m2 = jnp.dot(t, w_ref[...], preferred_element_type=F32) + b2_ref[...]
        m2_ref[...] = m2
        st = jnp.concatenate(
            [jnp.sum(m2, axis=0)[None], jnp.sum(m2 * m2, axis=0)[None],
             jnp.zeros((6, H), F32)], axis=0)
        _acc_stats(st2_ref, st, i)

    return pl.pallas_call(
        body,
        grid=(nblk,),
        in_specs=[pl.BlockSpec((_BN, H), lambda i: (i, 0)),
                  pl.BlockSpec((8, H), lambda i: (0, 0)),
                  pl.BlockSpec((1, H), lambda i: (0, 0)),
                  pl.BlockSpec((1, H), lambda i: (0, 0)),
                  pl.BlockSpec((H, H), lambda i: (0, 0)),
                  pl.BlockSpec((1, H), lambda i: (0, 0))],
        out_specs=[pl.BlockSpec((_BN, H), lambda i: (i, 0)),
                   pl.BlockSpec((8, H), lambda i: (0, 0))],
        out_shape=[jax.ShapeDtypeStruct((n, H), F32),
                   jax.ShapeDtypeStruct((8, H), F32)],
    )(m, st, g, b, w2, b2)


def _s3(m2, st2, g, b, npad):
    n, H = m2.shape
    C = H // _LANE
    nblk = n // _BN
    inv_n = 1.0 / n

    def body(m2_ref, st_ref, g_ref, b_ref, hc_ref, gp_ref):
        i = pl.program_id(0)
        scale, shift = _bn_affine(st_ref, g_ref, b_ref, inv_n)
        h = jnp.maximum(m2_ref[...] * scale + shift, 0.0)
        for c in range(C):
            hc_ref[c] = h[:, c * _LANE:(c + 1) * _LANE]
        st = jnp.concatenate(
            [jnp.sum(h, axis=0)[None], jnp.zeros((7, H), F32)], axis=0)
        _acc_stats(gp_ref, st, i)

    return pl.pallas_call(
        body,
        grid=(nblk,),
        in_specs=[pl.BlockSpec((_BN, H), lambda i: (i, 0)),
                  pl.BlockSpec((8, H), lambda i: (0, 0)),
                  pl.BlockSpec((1, H), lambda i: (0, 0)),
                  pl.BlockSpec((1, H), lambda i: (0, 0))],
        out_specs=[pl.BlockSpec((C, _BN, _LANE), lambda i: (0, i, 0)),
                   pl.BlockSpec((8, H), lambda i: (0, 0))],
        out_shape=[jax.ShapeDtypeStruct((C, npad, _LANE), F32),
                   jax.ShapeDtypeStruct((8, H), F32)],
    )(m2, st2, g, b)


def _head(gps, ws, bs):
    od = ws[0].shape[1]
    nl = len(ws)

    def body(*refs):
        gp_refs = refs[:nl]
        w_refs = refs[nl:2 * nl]
        b_refs = refs[2 * nl:3 * nl]
        out_ref = refs[3 * nl]
        tot = jnp.zeros((8, od), F32)
        bsum = jnp.zeros((1, od), F32)
        for gp, w, bb in zip(gp_refs, w_refs, b_refs):
            tot = tot + jnp.dot(gp[...], w[...], preferred_element_type=F32)
            bsum = bsum + bb[...]
        rows = lax.broadcasted_iota(jnp.int32, (8, od), 0)
        out_ref[...] = tot + jnp.where(rows == 0, bsum, 0.0)

    specs = ([pl.BlockSpec(g.shape, lambda i: (0, 0)) for g in gps]
             + [pl.BlockSpec(w.shape, lambda i: (0, 0)) for w in ws]
             + [pl.BlockSpec(b.shape, lambda i: (0, 0)) for b in bs])
    return pl.pallas_call(
        body,
        grid=(1,),
        in_specs=specs,
        out_specs=pl.BlockSpec((8, od), lambda i: (0, 0)),
        out_shape=jax.ShapeDtypeStruct((8, od), F32),
    )(*gps, *ws, *bs)


def _agg(hflat, gidx, sidx, C, npad, nb):
    return _make_agg(C, npad, nb)(hflat, gidx, sidx)


def kernel(x, edge_index, batch, params):
    n, din = x.shape
    e = edge_index.shape[1]
    convs = params['convs']
    preds = params['preds']

    nb = -(-e // (_TILES * _KB))
    epad = _TILES * nb * _KB
    npad = _npad(n)
    src = edge_index[0]
    dst = edge_index[1]
    dsta = jnp.concatenate(
        [dst, jnp.zeros((epad - e,), jnp.int32)]).reshape(_TILES, nb, _KB)
    sidx = jnp.concatenate(
        [src, jnp.full((epad - e,), n, jnp.int32)]).reshape(_TILES, nb, _KB)

    def gidx_for(C):
        offs = (jnp.arange(C, dtype=jnp.int32) * npad)[:, None, None, None]
        return dsta[None] + offs

    xc, gp0 = _k0(x, npad)
    gps = [gp0]
    hflat = xc.reshape(-1, _LANE)
    C = din // _LANE
    for cv in convs:
        pooled = _agg(hflat, gidx_for(C), sidx, C, npad, nb)
        hid = cv['W1'].shape[1]
        m, st = _s1(pooled, cv['W1'].reshape(C, _LANE, hid),
                    cv['b1'].reshape(1, -1), n)
        m2, st2 = _s2(m, st, cv['bn1_g'].reshape(1, -1),
                      cv['bn1_b'].reshape(1, -1), cv['W2'],
                      cv['b2'].reshape(1, -1))
        hc, gp = _s3(m2, st2, cv['bn_g'].reshape(1, -1),
                     cv['bn_b'].reshape(1, -1), npad)
        gps.append(gp)
        hflat = hc.reshape(-1, _LANE)
        C = hc.shape[0]

    score8 = _head(gps, [p['W'] for p in preds],
                   [p['b'].reshape(1, -1) for p in preds])
    return score8[0:1]


# speedup vs baseline: 1.0184x; 1.0012x over previous
import functools

import jax
import jax.numpy as jnp
from jax import lax
from jax.experimental import pallas as pl
from jax.experimental.pallas import tpu as pltpu
from jax.experimental.pallas import tpu_sc as plsc

F32 = jnp.float32

_LANE = 128
_KB = 128
_TILES = 16
_BN = 1000
_EPS = 1e-5


def _npad(n):
    rpt = -(-(-(-n // _TILES)) // 8) * 8
    while rpt * _TILES <= n:
        rpt += 8
    return rpt * _TILES


@functools.lru_cache(maxsize=None)
def _make_agg(C, npad, nb):
    rpt = npad // _TILES
    cpc = C // 2
    mesh = plsc.VectorSubcoreMesh(core_axis_name="c", subcore_axis_name="s")

    def body(hflat, gidx, sidx, out, gidx_v, sidx_v, rows_v, acc, sem):
        cid = lax.axis_index("c")
        sid = lax.axis_index("s")
        r0 = sid * rpt
        pltpu.sync_copy(sidx.at[sid], sidx_v)
        for cc in range(cpc):
            c = cid * cpc + cc
            pltpu.sync_copy(hflat.at[pl.ds(c * npad + r0, rpt)],
                            acc.at[pl.ds(r0, rpt)])
            pltpu.sync_copy(gidx.at[c, sid], gidx_v)
            plsc.subcore_barrier()

            @pl.loop(0, nb)
            def _edges(j):
                pltpu.async_copy(hflat.at[gidx_v.at[j]], rows_v, sem).wait()
                pltpu.sync_copy(rows_v, acc.at[sidx_v.at[j]], add=True)

            plsc.subcore_barrier()
            pltpu.sync_copy(acc.at[pl.ds(r0, rpt)], out.at[c, pl.ds(r0, rpt)])
            plsc.subcore_barrier()

    return pl.kernel(
        body,
        out_type=jax.ShapeDtypeStruct((C, npad, _LANE), F32),
        mesh=mesh,
        scratch_types=[
            pltpu.VMEM((nb, _KB), jnp.int32),
            pltpu.VMEM((nb, _KB), jnp.int32),
            pltpu.VMEM((_KB, _LANE), F32),
            pltpu.VMEM_SHARED((npad, _LANE), F32),
            pltpu.SemaphoreType.DMA,
        ],
    )


def _acc_stats(ref, st, i):
    @pl.when(i == 0)
    def _():
        ref[...] = st

    @pl.when(i > 0)
    def _():
        ref[...] = ref[...] + st


def _k0(x, npad):
    n, din = x.shape
    C = din // _LANE
    nblk = n // _BN

    def body(x_ref, xc_ref, gp_ref):
        i = pl.program_id(0)
        xb = x_ref[...]
        for c in range(C):
            xc_ref[c] = xb[:, c * _LANE:(c + 1) * _LANE]
        st = jnp.concatenate(
            [jnp.sum(xb, axis=0)[None], jnp.zeros((7, din), F32)], axis=0)
        _acc_stats(gp_ref, st, i)

    return pl.pallas_call(
        body,
        grid=(nblk,),
        in_specs=[pl.BlockSpec((_BN, din), lambda i: (i, 0))],
        out_specs=[pl.BlockSpec((C, _BN, _LANE), lambda i: (0, i, 0)),
                   pl.BlockSpec((8, din), lambda i: (0, 0))],
        out_shape=[jax.ShapeDtypeStruct((C, npad, _LANE), F32),
                   jax.ShapeDtypeStruct((8, din), F32)],
    )(x)


def _s1(pooled, w1r, b1, n):
    C, _, wc = pooled.shape
    H = w1r.shape[2]
    nblk = n // _BN

    def body(p_ref, w_ref, b_ref, m_ref, st_ref):
        i = pl.program_id(0)
        acc = jnp.zeros((_BN, H), F32)
        for c in range(C):
            acc = acc + jnp.dot(p_ref[c], w_ref[c], preferred_element_type=F32)
        m = acc + b_ref[...]
        m_ref[...] = m
        st = jnp.concatenate(
            [jnp.sum(m, axis=0)[None], jnp.sum(m * m, axis=0)[None],
             jnp.zeros((6, H), F32)], axis=0)
        _acc_stats(st_ref, st, i)

    return pl.pallas_call(
        body,
        grid=(nblk,),
        in_specs=[pl.BlockSpec((C, _BN, wc), lambda i: (0, i, 0)),
                  pl.BlockSpec((C, wc, H), lambda i: (0, 0, 0)),
                  pl.BlockSpec((1, H), lambda i: (0, 0))],
        out_specs=[pl.BlockSpec((_BN, H), lambda i: (i, 0)),
                   pl.BlockSpec((8, H), lambda i: (0, 0))],
        out_shape=[jax.ShapeDtypeStruct((n, H), F32),
                   jax.ShapeDtypeStruct((8, H), F32)],
    )(pooled, w1r, b1)


def _bn_affine(st_ref, g_ref, b_ref, inv_n):
    mean = st_ref[0:1, :] * inv_n
    var = st_ref[1:2, :] * inv_n - mean * mean
    scale = g_ref[...] * lax.rsqrt(var + _EPS)
    shift = b_ref[...] - mean * scale
    return scale, shift


def _s2(m, st, g, b, w2, b2):
    n, H = m.shape
    nblk = n // _BN
    inv_n = 1.0 / n

    def body(m_ref, st_ref, g_ref, b_ref, w_ref, b2_ref, m2_ref, st2_ref):
        i = pl.program_id(0)
        scale, shift = _bn_affine(st_ref, g_ref, b_ref, inv_n)
        t = jnp.maximum(m_ref[...] * scale + shift, 0.0)
        m2 = jnp.dot(t, w_ref[...], preferred_element_type=F32) + b2_ref[...]
        m2_ref[...] = m2
        st = jnp.concatenate(
            [jnp.sum(m2, axis=0)[None], jnp.sum(m2 * m2, axis=0)[None],
             jnp.zeros((6, H), F32)], axis=0)
        _acc_stats(st2_ref, st, i)

    return pl.pallas_call(
        body,
        grid=(nblk,),
        in_specs=[pl.BlockSpec((_BN, H), lambda i: (i, 0)),
                  pl.BlockSpec((8, H), lambda i: (0, 0)),
                  pl.BlockSpec((1, H), lambda i: (0, 0)),
                  pl.BlockSpec((1, H), lambda i: (0, 0)),
                  pl.BlockSpec((H, H), lambda i: (0, 0)),
                  pl.BlockSpec((1, H), lambda i: (0, 0))],
        out_specs=[pl.BlockSpec((_BN, H), lambda i: (i, 0)),
                   pl.BlockSpec((8, H), lambda i: (0, 0))],
        out_shape=[jax.ShapeDtypeStruct((n, H), F32),
                   jax.ShapeDtypeStruct((8, H), F32)],
    )(m, st, g, b, w2, b2)


def _s3(m2, st2, g, b, npad):
    n, H = m2.shape
    C = H // _LANE
    nblk = n // _BN
    inv_n = 1.0 / n

    def body(m2_ref, st_ref, g_ref, b_ref, hc_ref, gp_ref):
        i = pl.program_id(0)
        scale, shift = _bn_affine(st_ref, g_ref, b_ref, inv_n)
        h = jnp.maximum(m2_ref[...] * scale + shift, 0.0)
        for c in range(C):
            hc_ref[c] = h[:, c * _LANE:(c + 1) * _LANE]
        st = jnp.concatenate(
            [jnp.sum(h, axis=0)[None], jnp.zeros((7, H), F32)], axis=0)
        _acc_stats(gp_ref, st, i)

    return pl.pallas_call(
        body,
        grid=(nblk,),
        in_specs=[pl.BlockSpec((_BN, H), lambda i: (i, 0)),
                  pl.BlockSpec((8, H), lambda i: (0, 0)),
                  pl.BlockSpec((1, H), lambda i: (0, 0)),
                  pl.BlockSpec((1, H), lambda i: (0, 0))],
        out_specs=[pl.BlockSpec((C, _BN, _LANE), lambda i: (0, i, 0)),
                   pl.BlockSpec((8, H), lambda i: (0, 0))],
        out_shape=[jax.ShapeDtypeStruct((C, npad, _LANE), F32),
                   jax.ShapeDtypeStruct((8, H), F32)],
    )(m2, st2, g, b)


def _head(gps, ws, bs):
    od = ws[0].shape[1]
    nl = len(ws)

    def body(*refs):
        gp_refs = refs[:nl]
        w_refs = refs[nl:2 * nl]
        b_refs = refs[2 * nl:3 * nl]
        out_ref = refs[3 * nl]
        tot = jnp.zeros((8, od), F32)
        bsum = jnp.zeros((1, od), F32)
        for gp, w, bb in zip(gp_refs, w_refs, b_refs):
            tot = tot + jnp.dot(gp[...], w[...], preferred_element_type=F32)
            bsum = bsum + bb[...]
        rows = lax.broadcasted_iota(jnp.int32, (8, od), 0)
        out_ref[...] = tot + jnp.where(rows == 0, bsum, 0.0)

    specs = ([pl.BlockSpec(g.shape, lambda i: (0, 0)) for g in gps]
             + [pl.BlockSpec(w.shape, lambda i: (0, 0)) for w in ws]
             + [pl.BlockSpec(b.shape, lambda i: (0, 0)) for b in bs])
    return pl.pallas_call(
        body,
        grid=(1,),
        in_specs=specs,
        out_specs=pl.BlockSpec((8, od), lambda i: (0, 0)),
        out_shape=jax.ShapeDtypeStruct((8, od), F32),
    )(*gps, *ws, *bs)


def _agg(hflat, gidx, sidx, C, npad, nb):
    return _make_agg(C, npad, nb)(hflat, gidx, sidx)


def kernel(x, edge_index, batch, params):
    n, din = x.shape
    e = edge_index.shape[1]
    convs = params['convs']
    preds = params['preds']

    nb = -(-e // (_TILES * _KB))
    epad = _TILES * nb * _KB
    npad = _npad(n)
    src = edge_index[0]
    dst = edge_index[1]
    dsta = jnp.concatenate(
        [dst, jnp.zeros((epad - e,), jnp.int32)]).reshape(_TILES, nb, _KB)
    sidx = jnp.concatenate(
        [src, jnp.full((epad - e,), n, jnp.int32)]).reshape(_TILES, nb, _KB)

    def gidx_for(C):
        offs = (jnp.arange(C, dtype=jnp.int32) * npad)[:, None, None, None]
        return dsta[None] + offs

    xc, gp0 = _k0(x, npad)
    gps = [gp0]
    hflat = xc.reshape(-1, _LANE)
    C = din // _LANE
    for cv in convs:
        pooled = _agg(hflat, gidx_for(C), sidx, C, npad, nb)
        hid = cv['W1'].shape[1]
        m, st = _s1(pooled, cv['W1'].reshape(C, _LANE, hid),
                    cv['b1'].reshape(1, -1), n)
        m2, st2 = _s2(m, st, cv['bn1_g'].reshape(1, -1),
                      cv['bn1_b'].reshape(1, -1), cv['W2'],
                      cv['b2'].reshape(1, -1))
        hc, gp = _s3(m2, st2, cv['bn_g'].reshape(1, -1),
                     cv['bn_b'].reshape(1, -1), npad)
        gps.append(gp)
        hflat = hc.reshape(-1, _LANE)
        C = hc.shape[0]

    score8 = _head(gps, [p['W'] for p in preds],
                   [p['b'].reshape(1, -1) for p in preds])
    return score8[0:1]
